# baseline (speedup 1.0000x reference)
import functools

import jax
import jax.numpy as jnp
from jax import lax
from jax.experimental import pallas as pl
from jax.experimental.pallas import tpu as pltpu
from jax.experimental.pallas import tpu_sc as plsc

D = 768
NC, NS = 2, 16
NW = NC * NS
CHUNK = 32
NBUF = 5


@functools.lru_cache(maxsize=None)
def _build(B):
    rows_per_w = B // NW
    n_chunks = rows_per_w // CHUNK
    assert n_chunks % NBUF == 0
    mesh = plsc.VectorSubcoreMesh(core_axis_name="c", subcore_axis_name="s")

    @functools.partial(
        pl.kernel,
        mesh=mesh,
        out_type=jax.ShapeDtypeStruct((B, D), jnp.float32),
        scratch_types=[
            pltpu.VMEM((n_chunks, CHUNK), jnp.int32),
            pltpu.VMEM((NBUF, CHUNK, D), jnp.float32),
            pltpu.SemaphoreType.DMA((NBUF,)),
            pltpu.SemaphoreType.DMA((NBUF,)),
        ],
    )
    def gather_kernel(table_hbm, idx_hbm, out_hbm, idx_v, bufs, gsem, osem):
        wid = lax.axis_index("s") * NC + lax.axis_index("c")
        base = wid * rows_per_w
        pltpu.sync_copy(idx_hbm.at[wid], idx_v)

        def start_gather(j, b):
            pltpu.async_copy(table_hbm.at[idx_v.at[j]], bufs.at[b], gsem.at[b])

        def wait_gather(b):
            pltpu.make_async_copy(
                out_hbm.at[pl.ds(0, CHUNK)], bufs.at[b], gsem.at[b]
            ).wait()

        def start_write(j, b):
            pltpu.async_copy(
                bufs.at[b], out_hbm.at[pl.ds(base + j * CHUNK, CHUNK)], osem.at[b]
            )

        def wait_write(b):
            pltpu.make_async_copy(
                bufs.at[b], out_hbm.at[pl.ds(0, CHUNK)], osem.at[b]
            ).wait()

        for b in range(NBUF):
            start_gather(b, b)

        AHEAD = NBUF - 2

        def ring(p, carry):
            for b in range(NBUF):
                j = p * NBUF + b
                wait_gather(b)
                start_write(j, b)
                b2 = (b + AHEAD) % NBUF

                @pl.when((j >= 2) & (j + AHEAD < n_chunks))
                def _():
                    wait_write(b2)
                    start_gather(j + AHEAD, b2)

            return carry

        lax.fori_loop(0, n_chunks // NBUF, ring, 0)
        for b in range(NBUF):
            wait_write(b)

    return gather_kernel


def kernel(tokens, table):
    BT, T = tokens.shape
    B = BT * T
    idx = tokens.T.astype(jnp.int32).reshape(NW, B // (NW * CHUNK), CHUNK)
    out = _build(B)(table, idx)
    return out.reshape(T, BT, D).transpose(1, 0, 2)
